# baseline (speedup 1.0000x reference)
import functools

import jax
import jax.numpy as jnp
from jax import lax
from jax.experimental import pallas as pl
from jax.experimental.pallas import tpu as pltpu


def _mm_kernel(x_ref, w1_ref, b1_ref, w2_ref, out_ref, acc_ref, *, nk):
    j = pl.program_id(1)
    k = pl.program_id(2)

    part = jnp.dot(x_ref[...], w1_ref[...], preferred_element_type=jnp.float32)

    @pl.when(k == 0)
    def _():
        acc_ref[...] = part

    @pl.when(k > 0)
    def _():
        acc_ref[...] = acc_ref[...] + part

    @pl.when(k == nk - 1)
    def _():
        h = jnp.maximum(acc_ref[...] + b1_ref[...], 0.0)
        contrib = jnp.dot(h, w2_ref[...], preferred_element_type=jnp.float32)

        @pl.when(j == 0)
        def _():
            out_ref[...] = contrib

        @pl.when(j > 0)
        def _():
            out_ref[...] = out_ref[...] + contrib


def _router_logits(x, w1, b1r, w2p, *, bi, bj, bk):
    s, h = x.shape
    ni, nj, nk = s // bi, h // bj, h // bk
    return pl.pallas_call(
        functools.partial(_mm_kernel, nk=nk),
        grid=(ni, nj, nk),
        in_specs=[
            pl.BlockSpec((bi, bk), lambda i, j, k: (i, k)),
            pl.BlockSpec((bk, bj), lambda i, j, k: (k, j)),
            pl.BlockSpec((1, bj), lambda i, j, k: (0, j)),
            pl.BlockSpec((bj, 128), lambda i, j, k: (j, 0)),
        ],
        out_specs=pl.BlockSpec((bi, 128), lambda i, j, k: (i, 0)),
        out_shape=jax.ShapeDtypeStruct((s, 128), jnp.float32),
        scratch_shapes=[pltpu.VMEM((bi, bj), jnp.float32)],
    )(x, w1, b1r, w2p)


def _route_kernel(logits_ref, b2_ref, probs_ref, disp_ref, comb_ref, aux_ref,
                  carry_ref, psum_ref, *, nb, bs, E, CAP, S, K):
    b = pl.program_id(0)

    @pl.when(b == 0)
    def _():
        carry_ref[...] = jnp.zeros_like(carry_ref)
        psum_ref[...] = jnp.zeros_like(psum_ref)

    lane = lax.broadcasted_iota(jnp.int32, (bs, 128), 1)
    valid = lane < E

    logit = jnp.where(valid, logits_ref[...] + b2_ref[...], -1e30)
    m = jnp.max(logit, axis=1, keepdims=True)
    ex = jnp.where(valid, jnp.exp(logit - m), 0.0)
    denom = jnp.sum(ex, axis=1, keepdims=True)
    probs = ex / denom
    probs_ref[...] = probs

    v0 = jnp.max(probs, axis=1, keepdims=True)
    idx0 = jnp.min(jnp.where(probs == v0, lane, 127), axis=1, keepdims=True)
    masked = jnp.where(lane == idx0, -1.0, probs)
    v1 = jnp.max(masked, axis=1, keepdims=True)
    idx1 = jnp.min(jnp.where(masked == v1, lane, 127), axis=1, keepdims=True)

    nrm = v0 + v1 + 1e-8
    p0 = v0 / nrm
    p1 = v1 / nrm

    oh0 = (lane == idx0).astype(jnp.float32)
    oh1 = (lane == idx1).astype(jnp.float32)
    rowcnt = oh0 + oh1

    r_io = lax.broadcasted_iota(jnp.int32, (bs, bs), 0)
    c_io = lax.broadcasted_iota(jnp.int32, (bs, bs), 1)
    tri = (c_io < r_io).astype(jnp.float32)
    excl = jnp.dot(tri, rowcnt, preferred_element_type=jnp.float32)
    excl = excl + carry_ref[...]

    pos0 = jnp.sum(excl * oh0, axis=1, keepdims=True).astype(jnp.int32)
    pos1 = jnp.sum(excl * oh1, axis=1, keepdims=True).astype(jnp.int32)

    carry_ref[...] = carry_ref[...] + jnp.sum(rowcnt, axis=0, keepdims=True)
    psum_ref[...] = psum_ref[...] + jnp.sum(probs, axis=0, keepdims=True)

    flat0 = jnp.where(pos0 < CAP, idx0 * CAP + pos0, -1)
    flat1 = jnp.where(pos1 < CAP, idx1 * CAP + pos1, -1)

    col = lax.broadcasted_iota(jnp.int32, (bs, E * CAP), 1)
    hit0 = col == flat0
    hit1 = col == flat1
    disp_ref[...] = hit0.astype(jnp.float32) + hit1.astype(jnp.float32)
    comb_ref[...] = (jnp.where(hit0, p0, 0.0) + jnp.where(hit1, p1, 0.0))

    @pl.when(b == nb - 1)
    def _():
        usage = carry_ref[...] / float(S * K)
        pmean = psum_ref[...] / float(S)
        aux_ref[...] = jnp.sum(usage * pmean).reshape(1, 1) * float(E)


def _route(logits, b2p, *, bs, E, CAP, S, K):
    nb = S // bs
    return pl.pallas_call(
        functools.partial(_route_kernel, nb=nb, bs=bs, E=E, CAP=CAP, S=S, K=K),
        grid=(nb,),
        in_specs=[
            pl.BlockSpec((bs, 128), lambda b: (b, 0)),
            pl.BlockSpec((1, 128), lambda b: (0, 0)),
        ],
        out_specs=[
            pl.BlockSpec((bs, 128), lambda b: (b, 0)),
            pl.BlockSpec((bs, E * CAP), lambda b: (b, 0)),
            pl.BlockSpec((bs, E * CAP), lambda b: (b, 0)),
            pl.BlockSpec((1, 1), lambda b: (0, 0)),
        ],
        out_shape=[
            jax.ShapeDtypeStruct((S, 128), jnp.float32),
            jax.ShapeDtypeStruct((S, E * CAP), jnp.float32),
            jax.ShapeDtypeStruct((S, E * CAP), jnp.float32),
            jax.ShapeDtypeStruct((1, 1), jnp.float32),
        ],
        scratch_shapes=[
            pltpu.VMEM((1, 128), jnp.float32),
            pltpu.VMEM((1, 128), jnp.float32),
        ],
    )(logits, b2p)


def kernel(hidden_states, W1, b1, W2, b2):
    B, S, H = hidden_states.shape
    E = W2.shape[1]
    K = 2
    CF = 1.5
    CAP = int(B * S * CF * K / E)

    x = hidden_states.reshape(B * S, H)
    w2p = jnp.pad(W2, ((0, 0), (0, 128 - E)))
    b2p = jnp.pad(b2, (0, 128 - E)).reshape(1, 128)
    b1r = b1.reshape(1, H)

    logits = _router_logits(x, W1, b1r, w2p, bi=512, bj=512, bk=512)
    probs_p, disp, comb, aux = _route(logits, b2p, bs=128,
                                      E=E, CAP=CAP, S=B * S, K=K)

    dispatch = disp.reshape(B, S, E, CAP)
    combine = comb.reshape(B, S, E, CAP)
    router_probs = probs_p[:, :E].reshape(B, S, E)
    return dispatch, combine, router_probs, aux[0, 0]


# speedup vs baseline: 1.2372x; 1.2372x over previous
import functools

import jax
import jax.numpy as jnp
from jax import lax
from jax.experimental import pallas as pl
from jax.experimental.pallas import tpu as pltpu


def _mm_kernel(x_ref, w1_ref, b1_ref, w2_ref, out_ref, acc_ref, *, nk):
    j = pl.program_id(1)
    k = pl.program_id(2)

    part = jnp.dot(x_ref[...], w1_ref[...], preferred_element_type=jnp.float32)

    @pl.when(k == 0)
    def _():
        acc_ref[...] = part

    @pl.when(k > 0)
    def _():
        acc_ref[...] = acc_ref[...] + part

    @pl.when(k == nk - 1)
    def _():
        h = jnp.maximum(acc_ref[...] + b1_ref[...], 0.0)
        contrib = jnp.dot(h, w2_ref[...], preferred_element_type=jnp.float32)

        @pl.when(j == 0)
        def _():
            out_ref[...] = contrib

        @pl.when(j > 0)
        def _():
            out_ref[...] = out_ref[...] + contrib


def _router_logits(x, w1, b1r, w2p, *, bi, bj, bk):
    s, h = x.shape
    ni, nj, nk = s // bi, h // bj, h // bk
    return pl.pallas_call(
        functools.partial(_mm_kernel, nk=nk),
        grid=(ni, nj, nk),
        in_specs=[
            pl.BlockSpec((bi, bk), lambda i, j, k: (i, k)),
            pl.BlockSpec((bk, bj), lambda i, j, k: (k, j)),
            pl.BlockSpec((1, bj), lambda i, j, k: (0, j)),
            pl.BlockSpec((bj, 128), lambda i, j, k: (j, 0)),
        ],
        out_specs=pl.BlockSpec((bi, 128), lambda i, j, k: (i, 0)),
        out_shape=jax.ShapeDtypeStruct((s, 128), jnp.float32),
        scratch_shapes=[pltpu.VMEM((bi, bj), jnp.float32)],
    )(x, w1, b1r, w2p)


def _route_kernel(logits_ref, b2_ref, probs_ref, disp_ref, comb_ref, aux_ref,
                  carry_ref, psum_ref, *, nb, bs, E, CAP, S, K):
    b = pl.program_id(0)

    @pl.when(b == 0)
    def _():
        carry_ref[...] = jnp.zeros_like(carry_ref)
        psum_ref[...] = jnp.zeros_like(psum_ref)

    lane = lax.broadcasted_iota(jnp.int32, (bs, 128), 1)
    valid = lane < E

    logit = jnp.where(valid, logits_ref[...] + b2_ref[...], -1e30)
    m = jnp.max(logit, axis=1, keepdims=True)
    ex = jnp.where(valid, jnp.exp(logit - m), 0.0)
    denom = jnp.sum(ex, axis=1, keepdims=True)
    probs = ex / denom
    probs_ref[...] = probs

    v0 = jnp.max(probs, axis=1, keepdims=True)
    idx0 = jnp.min(jnp.where(probs == v0, lane, 127), axis=1, keepdims=True)
    masked = jnp.where(lane == idx0, -1.0, probs)
    v1 = jnp.max(masked, axis=1, keepdims=True)
    idx1 = jnp.min(jnp.where(masked == v1, lane, 127), axis=1, keepdims=True)

    nrm = v0 + v1 + 1e-8
    p0 = v0 / nrm
    p1 = v1 / nrm

    oh0 = (lane == idx0).astype(jnp.float32)
    oh1 = (lane == idx1).astype(jnp.float32)
    rowcnt = oh0 + oh1

    r_io = lax.broadcasted_iota(jnp.int32, (bs, bs), 0)
    c_io = lax.broadcasted_iota(jnp.int32, (bs, bs), 1)
    tri = (c_io < r_io).astype(jnp.float32)
    excl = jnp.dot(tri, rowcnt, preferred_element_type=jnp.float32)
    excl = excl + carry_ref[...]

    pos0 = jnp.sum(excl * oh0, axis=1, keepdims=True).astype(jnp.int32)
    pos1 = jnp.sum(excl * oh1, axis=1, keepdims=True).astype(jnp.int32)

    carry_ref[...] = carry_ref[...] + jnp.sum(rowcnt, axis=0, keepdims=True)
    psum_ref[...] = psum_ref[...] + jnp.sum(probs, axis=0, keepdims=True)

    e_io = lax.broadcasted_iota(jnp.int32, (bs, E, CAP), 1)
    c_io = lax.broadcasted_iota(jnp.int32, (bs, E, CAP), 2)
    i0 = idx0.reshape(bs, 1, 1)
    i1 = idx1.reshape(bs, 1, 1)
    q0 = jnp.where(pos0 < CAP, pos0, -1).reshape(bs, 1, 1)
    q1 = jnp.where(pos1 < CAP, pos1, -1).reshape(bs, 1, 1)
    hit0 = (e_io == i0) & (c_io == q0)
    hit1 = (e_io == i1) & (c_io == q1)
    disp = hit0.astype(jnp.float32) + hit1.astype(jnp.float32)
    comb = (jnp.where(hit0, p0.reshape(bs, 1, 1), 0.0)
            + jnp.where(hit1, p1.reshape(bs, 1, 1), 0.0))
    disp_ref[...] = disp.reshape(1, bs, E, CAP)
    comb_ref[...] = comb.reshape(1, bs, E, CAP)

    @pl.when(b == nb - 1)
    def _():
        usage = carry_ref[...] / float(S * K)
        pmean = psum_ref[...] / float(S)
        aux_ref[...] = jnp.sum(usage * pmean).reshape(1, 1) * float(E)


def _route(logits, b2p, *, bs, E, CAP, S, K):
    nb = S // bs
    return pl.pallas_call(
        functools.partial(_route_kernel, nb=nb, bs=bs, E=E, CAP=CAP, S=S, K=K),
        grid=(nb,),
        in_specs=[
            pl.BlockSpec((bs, 128), lambda b: (b, 0)),
            pl.BlockSpec((1, 128), lambda b: (0, 0)),
        ],
        out_specs=[
            pl.BlockSpec((bs, 128), lambda b: (b, 0)),
            pl.BlockSpec((1, bs, E, CAP), lambda b: (0, b, 0, 0)),
            pl.BlockSpec((1, bs, E, CAP), lambda b: (0, b, 0, 0)),
            pl.BlockSpec((1, 1), lambda b: (0, 0)),
        ],
        out_shape=[
            jax.ShapeDtypeStruct((S, 128), jnp.float32),
            jax.ShapeDtypeStruct((1, S, E, CAP), jnp.float32),
            jax.ShapeDtypeStruct((1, S, E, CAP), jnp.float32),
            jax.ShapeDtypeStruct((1, 1), jnp.float32),
        ],
        scratch_shapes=[
            pltpu.VMEM((1, 128), jnp.float32),
            pltpu.VMEM((1, 128), jnp.float32),
        ],
    )(logits, b2p)


def kernel(hidden_states, W1, b1, W2, b2):
    B, S, H = hidden_states.shape
    E = W2.shape[1]
    K = 2
    CF = 1.5
    CAP = int(B * S * CF * K / E)

    x = hidden_states.reshape(B * S, H)
    w2p = jnp.pad(W2, ((0, 0), (0, 128 - E)))
    b2p = jnp.pad(b2, (0, 128 - E)).reshape(1, 128)
    b1r = b1.reshape(1, H)

    logits = _router_logits(x, W1, b1r, w2p, bi=512, bj=512, bk=512)
    probs_p, disp, comb, aux = _route(logits, b2p, bs=128,
                                      E=E, CAP=CAP, S=B * S, K=K)

    router_probs = probs_p[:, :E].reshape(B, S, E)
    return disp, comb, router_probs, aux[0, 0]


# speedup vs baseline: 2.5033x; 2.0233x over previous
import functools

import jax
import jax.numpy as jnp
from jax import lax
from jax.experimental import pallas as pl
from jax.experimental.pallas import tpu as pltpu


def _mm_kernel(x_ref, w1_ref, b1_ref, w2_ref, out_ref, acc_ref, *, nk):
    k = pl.program_id(0)
    j = pl.program_id(1)

    part = jnp.dot(x_ref[...], w1_ref[...], preferred_element_type=jnp.float32)

    @pl.when(k == 0)
    def _():
        acc_ref[j] = part

    @pl.when(k > 0)
    def _():
        acc_ref[j] = acc_ref[j] + part

    @pl.when(k == nk - 1)
    def _():
        h = jnp.maximum(acc_ref[j] + b1_ref[...], 0.0)
        contrib = jnp.dot(h, w2_ref[...], preferred_element_type=jnp.float32)

        @pl.when(j == 0)
        def _():
            out_ref[...] = contrib

        @pl.when(j > 0)
        def _():
            out_ref[...] = out_ref[...] + contrib


def _router_logits(x, w1, b1r, w2p, *, bj, bk):
    s, h = x.shape
    nj, nk = h // bj, h // bk
    return pl.pallas_call(
        functools.partial(_mm_kernel, nk=nk),
        grid=(nk, nj),
        in_specs=[
            pl.BlockSpec((s, bk), lambda k, j: (0, k)),
            pl.BlockSpec((bk, bj), lambda k, j: (k, j)),
            pl.BlockSpec((1, bj), lambda k, j: (0, j)),
            pl.BlockSpec((bj, 128), lambda k, j: (j, 0)),
        ],
        out_specs=pl.BlockSpec((s, 128), lambda k, j: (0, 0)),
        out_shape=jax.ShapeDtypeStruct((s, 128), jnp.float32),
        scratch_shapes=[pltpu.VMEM((nj, s, bj), jnp.float32)],
    )(x, w1, b1r, w2p)


def _route_kernel(logits_ref, b2_ref, probs_ref, disp_ref, comb_ref, aux_ref,
                  carry_ref, psum_ref, *, nb, bs, E, CAP, S, K):
    b = pl.program_id(0)

    @pl.when(b == 0)
    def _():
        carry_ref[...] = jnp.zeros_like(carry_ref)
        psum_ref[...] = jnp.zeros_like(psum_ref)

    lane = lax.broadcasted_iota(jnp.int32, (bs, 128), 1)
    valid = lane < E

    logit = jnp.where(valid, logits_ref[...] + b2_ref[...], -1e30)
    m = jnp.max(logit, axis=1, keepdims=True)
    ex = jnp.where(valid, jnp.exp(logit - m), 0.0)
    denom = jnp.sum(ex, axis=1, keepdims=True)
    probs = ex / denom
    probs_ref[...] = probs

    v0 = jnp.max(probs, axis=1, keepdims=True)
    idx0 = jnp.min(jnp.where(probs == v0, lane, 127), axis=1, keepdims=True)
    masked = jnp.where(lane == idx0, -1.0, probs)
    v1 = jnp.max(masked, axis=1, keepdims=True)
    idx1 = jnp.min(jnp.where(masked == v1, lane, 127), axis=1, keepdims=True)

    nrm = v0 + v1 + 1e-8
    p0 = v0 / nrm
    p1 = v1 / nrm

    oh0 = (lane == idx0).astype(jnp.float32)
    oh1 = (lane == idx1).astype(jnp.float32)
    rowcnt = oh0 + oh1

    r_io = lax.broadcasted_iota(jnp.int32, (bs, bs), 0)
    c_io = lax.broadcasted_iota(jnp.int32, (bs, bs), 1)
    tri = (c_io < r_io).astype(jnp.float32)
    excl = jnp.dot(tri, rowcnt, preferred_element_type=jnp.float32)
    excl = excl + carry_ref[...]

    pos0 = jnp.sum(excl * oh0, axis=1, keepdims=True).astype(jnp.int32)
    pos1 = jnp.sum(excl * oh1, axis=1, keepdims=True).astype(jnp.int32)

    carry_ref[...] = carry_ref[...] + jnp.sum(rowcnt, axis=0, keepdims=True)
    psum_ref[...] = psum_ref[...] + jnp.sum(probs, axis=0, keepdims=True)

    e_io = lax.broadcasted_iota(jnp.int32, (bs, E, CAP), 1)
    c_io = lax.broadcasted_iota(jnp.int32, (bs, E, CAP), 2)
    i0 = idx0.reshape(bs, 1, 1)
    i1 = idx1.reshape(bs, 1, 1)
    q0 = jnp.where(pos0 < CAP, pos0, -1).reshape(bs, 1, 1)
    q1 = jnp.where(pos1 < CAP, pos1, -1).reshape(bs, 1, 1)
    hit0 = (e_io == i0) & (c_io == q0)
    hit1 = (e_io == i1) & (c_io == q1)
    disp = hit0.astype(jnp.float32) + hit1.astype(jnp.float32)
    comb = (jnp.where(hit0, p0.reshape(bs, 1, 1), 0.0)
            + jnp.where(hit1, p1.reshape(bs, 1, 1), 0.0))
    disp_ref[...] = disp.reshape(1, bs, E, CAP)
    comb_ref[...] = comb.reshape(1, bs, E, CAP)

    @pl.when(b == nb - 1)
    def _():
        usage = carry_ref[...] / float(S * K)
        pmean = psum_ref[...] / float(S)
        aux_ref[...] = jnp.sum(usage * pmean).reshape(1, 1) * float(E)


def _route(logits, b2p, *, bs, E, CAP, S, K):
    nb = S // bs
    return pl.pallas_call(
        functools.partial(_route_kernel, nb=nb, bs=bs, E=E, CAP=CAP, S=S, K=K),
        grid=(nb,),
        in_specs=[
            pl.BlockSpec((bs, 128), lambda b: (b, 0)),
            pl.BlockSpec((1, 128), lambda b: (0, 0)),
        ],
        out_specs=[
            pl.BlockSpec((bs, 128), lambda b: (b, 0)),
            pl.BlockSpec((1, bs, E, CAP), lambda b: (0, b, 0, 0)),
            pl.BlockSpec((1, bs, E, CAP), lambda b: (0, b, 0, 0)),
            pl.BlockSpec((1, 1), lambda b: (0, 0)),
        ],
        out_shape=[
            jax.ShapeDtypeStruct((S, 128), jnp.float32),
            jax.ShapeDtypeStruct((1, S, E, CAP), jnp.float32),
            jax.ShapeDtypeStruct((1, S, E, CAP), jnp.float32),
            jax.ShapeDtypeStruct((1, 1), jnp.float32),
        ],
        scratch_shapes=[
            pltpu.VMEM((1, 128), jnp.float32),
            pltpu.VMEM((1, 128), jnp.float32),
        ],
    )(logits, b2p)


def kernel(hidden_states, W1, b1, W2, b2):
    B, S, H = hidden_states.shape
    E = W2.shape[1]
    K = 2
    CF = 1.5
    CAP = int(B * S * CF * K / E)

    x = hidden_states.reshape(B * S, H)
    w2p = jnp.pad(W2, ((0, 0), (0, 128 - E)))
    b2p = jnp.pad(b2, (0, 128 - E)).reshape(1, 128)
    b1r = b1.reshape(1, H)

    logits = _router_logits(x, W1, b1r, w2p, bj=512, bk=512)
    probs_p, disp, comb, aux = _route(logits, b2p, bs=128,
                                      E=E, CAP=CAP, S=B * S, K=K)

    router_probs = probs_p[:, :E].reshape(B, S, E)
    return disp, comb, router_probs, aux[0, 0]


# speedup vs baseline: 2.7342x; 1.0922x over previous
import functools

import jax
import jax.numpy as jnp
from jax import lax
from jax.experimental import pallas as pl
from jax.experimental.pallas import tpu as pltpu


def _mm_kernel(x_ref, w1_ref, b1_ref, w2_ref, out_ref, acc_ref, *, nk):
    k = pl.program_id(0)
    j = pl.program_id(1)

    part = jnp.dot(x_ref[...], w1_ref[...], preferred_element_type=jnp.float32)

    @pl.when(k == 0)
    def _():
        acc_ref[j] = part

    @pl.when(k > 0)
    def _():
        acc_ref[j] = acc_ref[j] + part

    @pl.when(k == nk - 1)
    def _():
        h = jnp.maximum(acc_ref[j] + b1_ref[...], 0.0)
        contrib = jnp.dot(h, w2_ref[...], preferred_element_type=jnp.float32)

        @pl.when(j == 0)
        def _():
            out_ref[...] = contrib

        @pl.when(j > 0)
        def _():
            out_ref[...] = out_ref[...] + contrib


def _router_logits(x, w1, b1r, w2p, *, bj, bk):
    s, h = x.shape
    nj, nk = h // bj, h // bk
    return pl.pallas_call(
        functools.partial(_mm_kernel, nk=nk),
        grid=(nk, nj),
        in_specs=[
            pl.BlockSpec((s, bk), lambda k, j: (0, k)),
            pl.BlockSpec((bk, bj), lambda k, j: (k, j)),
            pl.BlockSpec((1, bj), lambda k, j: (0, j)),
            pl.BlockSpec((bj, 128), lambda k, j: (j, 0)),
        ],
        out_specs=pl.BlockSpec((s, 128), lambda k, j: (0, 0)),
        out_shape=jax.ShapeDtypeStruct((s, 128), jnp.float32),
        scratch_shapes=[pltpu.VMEM((nj, s, bj), jnp.float32)],
    )(x, w1, b1r, w2p)


def _route_kernel(logits_ref, b2_ref, probs_ref, disp_ref, comb_ref, aux_ref,
                  carry_ref, psum_ref, *, nb, bs, E, CAP, S, K):
    b = pl.program_id(0)

    @pl.when(b == 0)
    def _():
        carry_ref[...] = jnp.zeros_like(carry_ref)
        psum_ref[...] = jnp.zeros_like(psum_ref)

    lane = lax.broadcasted_iota(jnp.int32, (bs, 128), 1)
    valid = lane < E

    logit = jnp.where(valid, logits_ref[...] + b2_ref[...], -1e30)
    m = jnp.max(logit, axis=1, keepdims=True)
    ex = jnp.where(valid, jnp.exp(logit - m), 0.0)
    denom = jnp.sum(ex, axis=1, keepdims=True)
    probs = ex / denom
    probs_ref[...] = probs

    v0 = jnp.max(probs, axis=1, keepdims=True)
    idx0 = jnp.min(jnp.where(probs == v0, lane, 127), axis=1, keepdims=True)
    masked = jnp.where(lane == idx0, -1.0, probs)
    v1 = jnp.max(masked, axis=1, keepdims=True)
    idx1 = jnp.min(jnp.where(masked == v1, lane, 127), axis=1, keepdims=True)

    nrm = v0 + v1 + 1e-8
    p0 = v0 / nrm
    p1 = v1 / nrm

    oh0 = (lane == idx0).astype(jnp.float32)
    oh1 = (lane == idx1).astype(jnp.float32)
    rowcnt = oh0 + oh1

    r_io = lax.broadcasted_iota(jnp.int32, (bs, bs), 0)
    c_io = lax.broadcasted_iota(jnp.int32, (bs, bs), 1)
    tri = (c_io < r_io).astype(jnp.float32)
    excl = jnp.dot(tri, rowcnt, preferred_element_type=jnp.float32)
    excl = excl + carry_ref[...]

    pos0 = jnp.sum(excl * oh0, axis=1, keepdims=True).astype(jnp.int32)
    pos1 = jnp.sum(excl * oh1, axis=1, keepdims=True).astype(jnp.int32)

    carry_ref[...] = carry_ref[...] + jnp.sum(rowcnt, axis=0, keepdims=True)
    psum_ref[...] = psum_ref[...] + jnp.sum(probs, axis=0, keepdims=True)

    e_io = lax.broadcasted_iota(jnp.int32, (bs, E, CAP), 1)
    c_io = lax.broadcasted_iota(jnp.int32, (bs, E, CAP), 2)
    i0 = idx0.reshape(bs, 1, 1)
    i1 = idx1.reshape(bs, 1, 1)
    q0 = jnp.where(pos0 < CAP, pos0, -1).reshape(bs, 1, 1)
    q1 = jnp.where(pos1 < CAP, pos1, -1).reshape(bs, 1, 1)
    hit0 = (e_io == i0) & (c_io == q0)
    hit1 = (e_io == i1) & (c_io == q1)
    disp = hit0.astype(jnp.float32) + hit1.astype(jnp.float32)
    comb = (jnp.where(hit0, p0.reshape(bs, 1, 1), 0.0)
            + jnp.where(hit1, p1.reshape(bs, 1, 1), 0.0))
    disp_ref[...] = disp.reshape(1, bs, E, CAP)
    comb_ref[...] = comb.reshape(1, bs, E, CAP)

    @pl.when(b == nb - 1)
    def _():
        usage = carry_ref[...] / float(S * K)
        pmean = psum_ref[...] / float(S)
        aux_ref[...] = jnp.sum(usage * pmean).reshape(1, 1) * float(E)


def _route(logits, b2p, *, bs, E, CAP, S, K):
    nb = S // bs
    return pl.pallas_call(
        functools.partial(_route_kernel, nb=nb, bs=bs, E=E, CAP=CAP, S=S, K=K),
        grid=(nb,),
        in_specs=[
            pl.BlockSpec((bs, 128), lambda b: (b, 0)),
            pl.BlockSpec((1, 128), lambda b: (0, 0)),
        ],
        out_specs=[
            pl.BlockSpec((bs, 128), lambda b: (b, 0)),
            pl.BlockSpec((1, bs, E, CAP), lambda b: (0, b, 0, 0)),
            pl.BlockSpec((1, bs, E, CAP), lambda b: (0, b, 0, 0)),
            pl.BlockSpec((1, 1), lambda b: (0, 0)),
        ],
        out_shape=[
            jax.ShapeDtypeStruct((S, 128), jnp.float32),
            jax.ShapeDtypeStruct((1, S, E, CAP), jnp.float32),
            jax.ShapeDtypeStruct((1, S, E, CAP), jnp.float32),
            jax.ShapeDtypeStruct((1, 1), jnp.float32),
        ],
        scratch_shapes=[
            pltpu.VMEM((1, 128), jnp.float32),
            pltpu.VMEM((1, 128), jnp.float32),
        ],
    )(logits, b2p)


def kernel(hidden_states, W1, b1, W2, b2):
    B, S, H = hidden_states.shape
    E = W2.shape[1]
    K = 2
    CF = 1.5
    CAP = int(B * S * CF * K / E)

    x = hidden_states.reshape(B * S, H)
    w2p = jnp.pad(W2, ((0, 0), (0, 128 - E)))
    b2p = jnp.pad(b2, (0, 128 - E)).reshape(1, 128)
    b1r = b1.reshape(1, H)

    logits = _router_logits(x, W1, b1r, w2p, bj=1024, bk=512)
    probs_p, disp, comb, aux = _route(logits, b2p, bs=128,
                                      E=E, CAP=CAP, S=B * S, K=K)

    router_probs = probs_p[:, :E].reshape(B, S, E)
    return disp, comb, router_probs, aux[0, 0]


# speedup vs baseline: 3.0946x; 1.1318x over previous
import functools

import jax
import jax.numpy as jnp
from jax import lax
from jax.experimental import pallas as pl
from jax.experimental.pallas import tpu as pltpu


def _mm_kernel(x_ref, w1_ref, b1_ref, w2_ref, out_ref):
    j = pl.program_id(0)

    h = jnp.maximum(
        jnp.dot(x_ref[...], w1_ref[...], preferred_element_type=jnp.float32)
        + b1_ref[...], 0.0)
    contrib = jnp.dot(h, w2_ref[...], preferred_element_type=jnp.float32)

    @pl.when(j == 0)
    def _():
        out_ref[...] = contrib

    @pl.when(j > 0)
    def _():
        out_ref[...] = out_ref[...] + contrib


def _router_logits(x, w1, b1r, w2p, *, bj):
    s, h = x.shape
    nj = h // bj
    return pl.pallas_call(
        _mm_kernel,
        grid=(nj,),
        in_specs=[
            pl.BlockSpec((s, h), lambda j: (0, 0)),
            pl.BlockSpec((h, bj), lambda j: (0, j)),
            pl.BlockSpec((1, bj), lambda j: (0, j)),
            pl.BlockSpec((bj, 128), lambda j: (j, 0)),
        ],
        out_specs=pl.BlockSpec((s, 128), lambda j: (0, 0)),
        out_shape=jax.ShapeDtypeStruct((s, 128), jnp.float32),
    )(x, w1, b1r, w2p)


def _route_kernel(logits_ref, b2_ref, probs_ref, disp_ref, comb_ref, aux_ref,
                  carry_ref, psum_ref, *, nb, bs, E, CAP, S, K):
    b = pl.program_id(0)

    @pl.when(b == 0)
    def _():
        carry_ref[...] = jnp.zeros_like(carry_ref)
        psum_ref[...] = jnp.zeros_like(psum_ref)

    lane = lax.broadcasted_iota(jnp.int32, (bs, 128), 1)
    valid = lane < E

    logit = jnp.where(valid, logits_ref[...] + b2_ref[...], -1e30)
    m = jnp.max(logit, axis=1, keepdims=True)
    ex = jnp.where(valid, jnp.exp(logit - m), 0.0)
    denom = jnp.sum(ex, axis=1, keepdims=True)
    probs = ex / denom
    probs_ref[...] = probs

    v0 = jnp.max(probs, axis=1, keepdims=True)
    idx0 = jnp.min(jnp.where(probs == v0, lane, 127), axis=1, keepdims=True)
    masked = jnp.where(lane == idx0, -1.0, probs)
    v1 = jnp.max(masked, axis=1, keepdims=True)
    idx1 = jnp.min(jnp.where(masked == v1, lane, 127), axis=1, keepdims=True)

    nrm = v0 + v1 + 1e-8
    p0 = v0 / nrm
    p1 = v1 / nrm

    oh0 = (lane == idx0).astype(jnp.float32)
    oh1 = (lane == idx1).astype(jnp.float32)
    rowcnt = oh0 + oh1

    r_io = lax.broadcasted_iota(jnp.int32, (bs, bs), 0)
    c_io = lax.broadcasted_iota(jnp.int32, (bs, bs), 1)
    tri = (c_io < r_io).astype(jnp.float32)
    excl = jnp.dot(tri, rowcnt, preferred_element_type=jnp.float32)
    excl = excl + carry_ref[...]

    pos0 = jnp.sum(excl * oh0, axis=1, keepdims=True).astype(jnp.int32)
    pos1 = jnp.sum(excl * oh1, axis=1, keepdims=True).astype(jnp.int32)

    carry_ref[...] = carry_ref[...] + jnp.sum(rowcnt, axis=0, keepdims=True)
    psum_ref[...] = psum_ref[...] + jnp.sum(probs, axis=0, keepdims=True)

    e_io = lax.broadcasted_iota(jnp.int32, (bs, E, CAP), 1)
    c_io = lax.broadcasted_iota(jnp.int32, (bs, E, CAP), 2)
    i0 = idx0.reshape(bs, 1, 1)
    i1 = idx1.reshape(bs, 1, 1)
    q0 = jnp.where(pos0 < CAP, pos0, -1).reshape(bs, 1, 1)
    q1 = jnp.where(pos1 < CAP, pos1, -1).reshape(bs, 1, 1)
    hit0 = (e_io == i0) & (c_io == q0)
    hit1 = (e_io == i1) & (c_io == q1)
    disp = hit0.astype(jnp.float32) + hit1.astype(jnp.float32)
    comb = (jnp.where(hit0, p0.reshape(bs, 1, 1), 0.0)
            + jnp.where(hit1, p1.reshape(bs, 1, 1), 0.0))
    disp_ref[...] = disp.reshape(1, bs, E, CAP)
    comb_ref[...] = comb.reshape(1, bs, E, CAP)

    @pl.when(b == nb - 1)
    def _():
        usage = carry_ref[...] / float(S * K)
        pmean = psum_ref[...] / float(S)
        aux_ref[...] = jnp.sum(usage * pmean).reshape(1, 1) * float(E)


def _route(logits, b2p, *, bs, E, CAP, S, K):
    nb = S // bs
    return pl.pallas_call(
        functools.partial(_route_kernel, nb=nb, bs=bs, E=E, CAP=CAP, S=S, K=K),
        grid=(nb,),
        in_specs=[
            pl.BlockSpec((bs, 128), lambda b: (b, 0)),
            pl.BlockSpec((1, 128), lambda b: (0, 0)),
        ],
        out_specs=[
            pl.BlockSpec((bs, 128), lambda b: (b, 0)),
            pl.BlockSpec((1, bs, E, CAP), lambda b: (0, b, 0, 0)),
            pl.BlockSpec((1, bs, E, CAP), lambda b: (0, b, 0, 0)),
            pl.BlockSpec((1, 1), lambda b: (0, 0)),
        ],
        out_shape=[
            jax.ShapeDtypeStruct((S, 128), jnp.float32),
            jax.ShapeDtypeStruct((1, S, E, CAP), jnp.float32),
            jax.ShapeDtypeStruct((1, S, E, CAP), jnp.float32),
            jax.ShapeDtypeStruct((1, 1), jnp.float32),
        ],
        scratch_shapes=[
            pltpu.VMEM((1, 128), jnp.float32),
            pltpu.VMEM((1, 128), jnp.float32),
        ],
    )(logits, b2p)


def kernel(hidden_states, W1, b1, W2, b2):
    B, S, H = hidden_states.shape
    E = W2.shape[1]
    K = 2
    CF = 1.5
    CAP = int(B * S * CF * K / E)

    x = hidden_states.reshape(B * S, H)
    w2p = jnp.pad(W2, ((0, 0), (0, 128 - E)))
    b2p = jnp.pad(b2, (0, 128 - E)).reshape(1, 128)
    b1r = b1.reshape(1, H)

    logits = _router_logits(x, W1, b1r, w2p, bj=512)
    probs_p, disp, comb, aux = _route(logits, b2p, bs=128,
                                      E=E, CAP=CAP, S=B * S, K=K)

    router_probs = probs_p[:, :E].reshape(B, S, E)
    return disp, comb, router_probs, aux[0, 0]


# speedup vs baseline: 3.2799x; 1.0599x over previous
import functools

import jax
import jax.numpy as jnp
from jax import lax
from jax.experimental import pallas as pl
from jax.experimental.pallas import tpu as pltpu


def _mm_kernel(x_ref, w1_ref, b1_ref, w2_ref, out_ref):
    j = pl.program_id(0)

    h = jnp.maximum(
        jnp.dot(x_ref[...], w1_ref[...], preferred_element_type=jnp.float32)
        + b1_ref[...], 0.0)
    contrib = jnp.dot(h, w2_ref[...], preferred_element_type=jnp.float32)

    @pl.when(j == 0)
    def _():
        out_ref[...] = contrib

    @pl.when(j > 0)
    def _():
        out_ref[...] = out_ref[...] + contrib


def _router_logits(x, w1, b1r, w2p, *, bj):
    s, h = x.shape
    nj = h // bj
    return pl.pallas_call(
        _mm_kernel,
        grid=(nj,),
        in_specs=[
            pl.BlockSpec((s, h), lambda j: (0, 0)),
            pl.BlockSpec((h, bj), lambda j: (0, j)),
            pl.BlockSpec((1, bj), lambda j: (0, j)),
            pl.BlockSpec((bj, 128), lambda j: (j, 0)),
        ],
        out_specs=pl.BlockSpec((s, 128), lambda j: (0, 0)),
        out_shape=jax.ShapeDtypeStruct((s, 128), jnp.float32),
    )(x, w1, b1r, w2p)


def _route_kernel(logits_ref, b2_ref, probs_ref, disp_ref, comb_ref, aux_ref,
                  carry_ref, psum_ref, *, nb, bs, E, CAP, S, K):
    b = pl.program_id(0)

    @pl.when(b == 0)
    def _():
        carry_ref[...] = jnp.zeros_like(carry_ref)
        psum_ref[...] = jnp.zeros_like(psum_ref)

    lane = lax.broadcasted_iota(jnp.int32, (bs, 128), 1)
    valid = lane < E

    logit = jnp.where(valid, logits_ref[...] + b2_ref[...], -1e30)
    m = jnp.max(logit, axis=1, keepdims=True)
    ex = jnp.where(valid, jnp.exp(logit - m), 0.0)
    denom = jnp.sum(ex, axis=1, keepdims=True)
    probs = ex / denom
    probs_ref[...] = probs

    v0 = jnp.max(probs, axis=1, keepdims=True)
    idx0 = jnp.min(jnp.where(probs == v0, lane, 127), axis=1, keepdims=True)
    masked = jnp.where(lane == idx0, -1.0, probs)
    v1 = jnp.max(masked, axis=1, keepdims=True)
    idx1 = jnp.min(jnp.where(masked == v1, lane, 127), axis=1, keepdims=True)

    nrm = v0 + v1 + 1e-8
    p0 = v0 / nrm
    p1 = v1 / nrm

    oh0 = (lane == idx0).astype(jnp.float32)
    oh1 = (lane == idx1).astype(jnp.float32)
    rowcnt = oh0 + oh1

    r_io = lax.broadcasted_iota(jnp.int32, (bs, bs), 0)
    c_io = lax.broadcasted_iota(jnp.int32, (bs, bs), 1)
    tri = (c_io < r_io).astype(jnp.float32)
    excl = jnp.dot(tri, rowcnt, preferred_element_type=jnp.float32)
    excl = excl + carry_ref[...]

    pos0 = jnp.sum(excl * oh0, axis=1, keepdims=True).astype(jnp.int32)
    pos1 = jnp.sum(excl * oh1, axis=1, keepdims=True).astype(jnp.int32)

    carry_ref[...] = carry_ref[...] + jnp.sum(rowcnt, axis=0, keepdims=True)
    psum_ref[...] = psum_ref[...] + jnp.sum(probs, axis=0, keepdims=True)

    e_io8 = lax.broadcasted_iota(jnp.int32, (bs, E, 1), 1)
    hit0e = e_io8 == idx0.reshape(bs, 1, 1)
    hit1e = e_io8 == idx1.reshape(bs, 1, 1)
    q0 = jnp.where(pos0 < CAP, pos0, -1).reshape(bs, 1, 1)
    q1 = jnp.where(pos1 < CAP, pos1, -1).reshape(bs, 1, 1)
    pos_e = jnp.where(hit0e, q0, jnp.where(hit1e, q1, -1))
    val_e = jnp.where(hit0e, p0.reshape(bs, 1, 1),
                      jnp.where(hit1e, p1.reshape(bs, 1, 1), 0.0))

    c_io = lax.broadcasted_iota(jnp.int32, (bs, E, CAP), 2)
    m = c_io == pos_e
    disp_ref[...] = m.astype(jnp.float32).reshape(1, bs, E, CAP)
    comb_ref[...] = jnp.where(m, val_e, 0.0).reshape(1, bs, E, CAP)

    @pl.when(b == nb - 1)
    def _():
        usage = carry_ref[...] / float(S * K)
        pmean = psum_ref[...] / float(S)
        aux_ref[...] = jnp.sum(usage * pmean).reshape(1, 1) * float(E)


def _route(logits, b2p, *, bs, E, CAP, S, K):
    nb = S // bs
    return pl.pallas_call(
        functools.partial(_route_kernel, nb=nb, bs=bs, E=E, CAP=CAP, S=S, K=K),
        grid=(nb,),
        in_specs=[
            pl.BlockSpec((bs, 128), lambda b: (b, 0)),
            pl.BlockSpec((1, 128), lambda b: (0, 0)),
        ],
        out_specs=[
            pl.BlockSpec((bs, 128), lambda b: (b, 0)),
            pl.BlockSpec((1, bs, E, CAP), lambda b: (0, b, 0, 0)),
            pl.BlockSpec((1, bs, E, CAP), lambda b: (0, b, 0, 0)),
            pl.BlockSpec((1, 1), lambda b: (0, 0)),
        ],
        out_shape=[
            jax.ShapeDtypeStruct((S, 128), jnp.float32),
            jax.ShapeDtypeStruct((1, S, E, CAP), jnp.float32),
            jax.ShapeDtypeStruct((1, S, E, CAP), jnp.float32),
            jax.ShapeDtypeStruct((1, 1), jnp.float32),
        ],
        scratch_shapes=[
            pltpu.VMEM((1, 128), jnp.float32),
            pltpu.VMEM((1, 128), jnp.float32),
        ],
    )(logits, b2p)


def kernel(hidden_states, W1, b1, W2, b2):
    B, S, H = hidden_states.shape
    E = W2.shape[1]
    K = 2
    CF = 1.5
    CAP = int(B * S * CF * K / E)

    x = hidden_states.reshape(B * S, H)
    w2p = jnp.pad(W2, ((0, 0), (0, 128 - E)))
    b2p = jnp.pad(b2, (0, 128 - E)).reshape(1, 128)
    b1r = b1.reshape(1, H)

    logits = _router_logits(x, W1, b1r, w2p, bj=512)
    probs_p, disp, comb, aux = _route(logits, b2p, bs=128,
                                      E=E, CAP=CAP, S=B * S, K=K)

    router_probs = probs_p[:, :E].reshape(B, S, E)
    return disp, comb, router_probs, aux[0, 0]


# speedup vs baseline: 3.3174x; 1.0114x over previous
import functools

import jax
import jax.numpy as jnp
from jax import lax
from jax.experimental import pallas as pl
from jax.experimental.pallas import tpu as pltpu


def _mm_kernel(x_ref, w1_ref, b1_ref, w2_ref, out_ref):
    j = pl.program_id(0)

    h = jnp.maximum(
        jnp.dot(x_ref[...], w1_ref[...], preferred_element_type=jnp.float32)
        + b1_ref[...], 0.0)
    contrib = jnp.dot(h, w2_ref[...], preferred_element_type=jnp.float32)

    @pl.when(j == 0)
    def _():
        out_ref[...] = contrib

    @pl.when(j > 0)
    def _():
        out_ref[...] = out_ref[...] + contrib


def _router_logits(x, w1, b1r, w2p, *, bj):
    s, h = x.shape
    nj = h // bj
    return pl.pallas_call(
        _mm_kernel,
        grid=(nj,),
        in_specs=[
            pl.BlockSpec((s, h), lambda j: (0, 0)),
            pl.BlockSpec((h, bj), lambda j: (0, j)),
            pl.BlockSpec((1, bj), lambda j: (0, j)),
            pl.BlockSpec((bj, 128), lambda j: (j, 0)),
        ],
        out_specs=pl.BlockSpec((s, 128), lambda j: (0, 0)),
        out_shape=jax.ShapeDtypeStruct((s, 128), jnp.float32),
    )(x, w1, b1r, w2p)


def _route_kernel(logits_ref, b2_ref, probs_ref, disp_ref, comb_ref, aux_ref,
                  carry_ref, psum_ref, *, nb, bs, E, CAP, S, K):
    b = pl.program_id(0)

    @pl.when(b == 0)
    def _():
        carry_ref[...] = jnp.zeros_like(carry_ref)
        psum_ref[...] = jnp.zeros_like(psum_ref)

    lane = lax.broadcasted_iota(jnp.int32, (bs, 128), 1)
    valid = lane < E

    logit = jnp.where(valid, logits_ref[...] + b2_ref[...], -1e30)
    m = jnp.max(logit, axis=1, keepdims=True)
    ex = jnp.where(valid, jnp.exp(logit - m), 0.0)
    denom = jnp.sum(ex, axis=1, keepdims=True)
    probs = ex / denom
    probs_ref[...] = probs

    v0 = jnp.max(probs, axis=1, keepdims=True)
    idx0 = jnp.min(jnp.where(probs == v0, lane, 127), axis=1, keepdims=True)
    masked = jnp.where(lane == idx0, -1.0, probs)
    v1 = jnp.max(masked, axis=1, keepdims=True)
    idx1 = jnp.min(jnp.where(masked == v1, lane, 127), axis=1, keepdims=True)

    nrm = v0 + v1 + 1e-8
    p0 = v0 / nrm
    p1 = v1 / nrm

    oh0 = (lane == idx0).astype(jnp.float32)
    oh1 = (lane == idx1).astype(jnp.float32)
    rowcnt = oh0 + oh1

    r_io = lax.broadcasted_iota(jnp.int32, (bs, bs), 0)
    c_io = lax.broadcasted_iota(jnp.int32, (bs, bs), 1)
    tri = (c_io < r_io).astype(jnp.float32)
    excl = jnp.dot(tri, rowcnt, preferred_element_type=jnp.float32)
    excl = excl + carry_ref[...]

    pos0 = jnp.sum(excl * oh0, axis=1, keepdims=True).astype(jnp.int32)
    pos1 = jnp.sum(excl * oh1, axis=1, keepdims=True).astype(jnp.int32)

    carry_ref[...] = carry_ref[...] + jnp.sum(rowcnt, axis=0, keepdims=True)
    psum_ref[...] = psum_ref[...] + jnp.sum(probs, axis=0, keepdims=True)

    e_io8 = lax.broadcasted_iota(jnp.int32, (bs, E, 1), 1)
    hit0e = e_io8 == idx0.reshape(bs, 1, 1)
    hit1e = e_io8 == idx1.reshape(bs, 1, 1)
    q0 = jnp.where(pos0 < CAP, pos0, -1).reshape(bs, 1, 1)
    q1 = jnp.where(pos1 < CAP, pos1, -1).reshape(bs, 1, 1)
    pos_e = jnp.where(hit0e, q0, jnp.where(hit1e, q1, -1))
    val_e = jnp.where(hit0e, p0.reshape(bs, 1, 1),
                      jnp.where(hit1e, p1.reshape(bs, 1, 1), 0.0))

    c_io = lax.broadcasted_iota(jnp.int32, (bs, E, CAP), 2)
    m = c_io == pos_e
    disp_ref[...] = m.astype(jnp.float32).reshape(1, bs, E, CAP)
    comb_ref[...] = jnp.where(m, val_e, 0.0).reshape(1, bs, E, CAP)

    @pl.when(b == nb - 1)
    def _():
        usage = carry_ref[...] / float(S * K)
        pmean = psum_ref[...] / float(S)
        aux_ref[...] = jnp.sum(usage * pmean).reshape(1, 1) * float(E)


def _route(logits, b2p, *, bs, E, CAP, S, K):
    nb = S // bs
    return pl.pallas_call(
        functools.partial(_route_kernel, nb=nb, bs=bs, E=E, CAP=CAP, S=S, K=K),
        grid=(nb,),
        in_specs=[
            pl.BlockSpec((bs, 128), lambda b: (b, 0)),
            pl.BlockSpec((1, 128), lambda b: (0, 0)),
        ],
        out_specs=[
            pl.BlockSpec((bs, 128), lambda b: (b, 0)),
            pl.BlockSpec((1, bs, E, CAP), lambda b: (0, b, 0, 0)),
            pl.BlockSpec((1, bs, E, CAP), lambda b: (0, b, 0, 0)),
            pl.BlockSpec((1, 1), lambda b: (0, 0)),
        ],
        out_shape=[
            jax.ShapeDtypeStruct((S, 128), jnp.float32),
            jax.ShapeDtypeStruct((1, S, E, CAP), jnp.float32),
            jax.ShapeDtypeStruct((1, S, E, CAP), jnp.float32),
            jax.ShapeDtypeStruct((1, 1), jnp.float32),
        ],
        scratch_shapes=[
            pltpu.VMEM((1, 128), jnp.float32),
            pltpu.VMEM((1, 128), jnp.float32),
        ],
    )(logits, b2p)


def kernel(hidden_states, W1, b1, W2, b2):
    B, S, H = hidden_states.shape
    E = W2.shape[1]
    K = 2
    CF = 1.5
    CAP = int(B * S * CF * K / E)

    x = hidden_states.reshape(B * S, H)
    w2p = jnp.pad(W2, ((0, 0), (0, 128 - E)))
    b2p = jnp.pad(b2, (0, 128 - E)).reshape(1, 128)
    b1r = b1.reshape(1, H)

    logits = _router_logits(x, W1, b1r, w2p, bj=512)
    probs_p, disp, comb, aux = _route(logits, b2p, bs=256,
                                      E=E, CAP=CAP, S=B * S, K=K)

    router_probs = probs_p[:, :E].reshape(B, S, E)
    return disp, comb, router_probs, aux[0, 0]
